# baseline (speedup 1.0000x reference)
import functools

import jax
import jax.numpy as jnp
from jax import lax
from jax.experimental import pallas as pl
from jax.experimental.pallas import tpu as pltpu
from jax.experimental.pallas import tpu_sc as plsc

B, S, V, D = 4, 8192, 100000, 768


def _make_lookup(n_rows: int, chunk: int):
    info = plsc.get_sparse_core_info()
    nc, ns = info.num_cores, info.num_subcores
    nw = nc * ns
    assert n_rows % (nw * chunk) == 0
    rows_per_w = n_rows // nw
    n_chunks = rows_per_w // chunk

    mesh = plsc.VectorSubcoreMesh(core_axis_name="c", subcore_axis_name="s")

    @functools.partial(
        pl.kernel,
        mesh=mesh,
        out_type=jax.ShapeDtypeStruct((n_rows, D), jnp.float32),
        scratch_types=[
            pltpu.VMEM((n_chunks, chunk), jnp.int32),
            pltpu.VMEM((chunk, D), jnp.float32),
            pltpu.SemaphoreType.DMA,
        ],
    )
    def lookup(table_hbm, idx_hbm, out_hbm, idx_v, rows_v, sem):
        wid = lax.axis_index("s") * nc + lax.axis_index("c")
        base = wid * rows_per_w
        pltpu.sync_copy(idx_hbm.at[wid], idx_v)
        for c in range(n_chunks):
            pltpu.async_copy(table_hbm.at[idx_v.at[c]], rows_v, sem).wait()
            pltpu.sync_copy(rows_v, out_hbm.at[pl.ds(base + c * chunk, chunk)])

    return lookup, nw, n_chunks


def kernel(enc_tokens, dec_tokens, enc_attn_mask, dec_attn_mask,
           enc_dec_attn_mask, dec_labels, emb_table):
    n_rows = B * S
    chunk = 128
    lookup, nw, n_chunks = _make_lookup(n_rows, chunk)
    idx = enc_tokens.T.astype(jnp.int32).reshape(nw, n_chunks, chunk)
    out = lookup(emb_table, idx)
    return out.reshape(S, B, D)


# speedup vs baseline: 1.0118x; 1.0118x over previous
import functools

import jax
import jax.numpy as jnp
from jax import lax
from jax.experimental import pallas as pl
from jax.experimental.pallas import tpu as pltpu
from jax.experimental.pallas import tpu_sc as plsc

B, S, V, D = 4, 8192, 100000, 768


def _make_lookup(n_rows: int, chunk: int):
    info = plsc.get_sparse_core_info()
    nc, ns = info.num_cores, info.num_subcores
    nw = nc * ns
    assert n_rows % (nw * chunk) == 0
    rows_per_w = n_rows // nw
    n_chunks = rows_per_w // chunk

    mesh = plsc.VectorSubcoreMesh(core_axis_name="c", subcore_axis_name="s")

    @functools.partial(
        pl.kernel,
        mesh=mesh,
        out_type=jax.ShapeDtypeStruct((n_rows, D), jnp.float32),
        scratch_types=[
            pltpu.VMEM((n_chunks, chunk), jnp.int32),
            pltpu.VMEM((chunk, D), jnp.float32),
            pltpu.VMEM((chunk, D), jnp.float32),
            pltpu.SemaphoreType.DMA,
            pltpu.SemaphoreType.DMA,
            pltpu.SemaphoreType.DMA,
            pltpu.SemaphoreType.DMA,
        ],
    )
    def lookup(table_hbm, idx_hbm, out_hbm, idx_v, rows_a, rows_b,
               gsem_a, gsem_b, wsem_a, wsem_b):
        wid = lax.axis_index("s") * nc + lax.axis_index("c")
        base = wid * rows_per_w
        pltpu.sync_copy(idx_hbm.at[wid], idx_v)
        bufs = (rows_a, rows_b)
        gsems = (gsem_a, gsem_b)
        wsems = (wsem_a, wsem_b)
        gathers = [None, None]
        writes = [None, None]
        gathers[0] = pltpu.async_copy(table_hbm.at[idx_v.at[0]], bufs[0],
                                      gsems[0])
        for c in range(n_chunks):
            b = c & 1
            nb = 1 - b
            if c + 1 < n_chunks:
                if writes[nb] is not None:
                    writes[nb].wait()
                gathers[nb] = pltpu.async_copy(
                    table_hbm.at[idx_v.at[c + 1]], bufs[nb], gsems[nb])
            gathers[b].wait()
            writes[b] = pltpu.async_copy(
                bufs[b], out_hbm.at[pl.ds(base + c * chunk, chunk)], wsems[b])
        for w in writes:
            if w is not None:
                w.wait()

    return lookup, nw, n_chunks


def kernel(enc_tokens, dec_tokens, enc_attn_mask, dec_attn_mask,
           enc_dec_attn_mask, dec_labels, emb_table):
    n_rows = B * S
    chunk = 64
    lookup, nw, n_chunks = _make_lookup(n_rows, chunk)
    idx = enc_tokens.T.astype(jnp.int32).reshape(nw, n_chunks, chunk)
    out = lookup(emb_table, idx)
    return out.reshape(S, B, D)


# speedup vs baseline: 2.1542x; 2.1291x over previous
import functools

import jax
import jax.numpy as jnp
from jax import lax
from jax.experimental import pallas as pl
from jax.experimental.pallas import tpu as pltpu
from jax.experimental.pallas import tpu_sc as plsc

B, S, V, D = 4, 8192, 100000, 768


def _make_lookup(chunk: int):
    info = plsc.get_sparse_core_info()
    nc, ns = info.num_cores, info.num_subcores
    nw = nc * ns
    n_slabs = nw // B
    slab = S // n_slabs
    assert slab % chunk == 0
    n_chunks = slab // chunk

    mesh = plsc.VectorSubcoreMesh(core_axis_name="c", subcore_axis_name="s")

    @functools.partial(
        pl.kernel,
        mesh=mesh,
        out_type=jax.ShapeDtypeStruct((S, B, D), jnp.float32),
        scratch_types=[
            pltpu.VMEM((n_chunks, chunk), jnp.int32),
            pltpu.VMEM((chunk, D), jnp.float32),
            pltpu.VMEM((chunk, D), jnp.float32),
            pltpu.SemaphoreType.DMA,
            pltpu.SemaphoreType.DMA,
            pltpu.SemaphoreType.DMA,
            pltpu.SemaphoreType.DMA,
        ],
    )
    def lookup(table_hbm, idx_hbm, out_hbm, idx_v, rows_a, rows_b,
               gsem_a, gsem_b, wsem_a, wsem_b):
        wid = lax.axis_index("s") * nc + lax.axis_index("c")
        b = wid // n_slabs
        sb = wid % n_slabs
        s_base = sb * slab
        pltpu.sync_copy(idx_hbm.at[b, sb], idx_v)
        bufs = (rows_a, rows_b)
        gsems = (gsem_a, gsem_b)
        wsems = (wsem_a, wsem_b)
        gathers = [None, None]
        writes = [None, None]
        gathers[0] = pltpu.async_copy(table_hbm.at[idx_v.at[0]], bufs[0],
                                      gsems[0])
        for c in range(n_chunks):
            cur = c & 1
            nxt = 1 - cur
            if c + 1 < n_chunks:
                if writes[nxt] is not None:
                    writes[nxt].wait()
                gathers[nxt] = pltpu.async_copy(
                    table_hbm.at[idx_v.at[c + 1]], bufs[nxt], gsems[nxt])
            gathers[cur].wait()
            writes[cur] = pltpu.async_copy(
                bufs[cur], out_hbm.at[pl.ds(s_base + c * chunk, chunk), b, :],
                wsems[cur])
        for w in writes:
            if w is not None:
                w.wait()

    return lookup, n_slabs, n_chunks


def kernel(enc_tokens, dec_tokens, enc_attn_mask, dec_attn_mask,
           enc_dec_attn_mask, dec_labels, emb_table):
    chunk = 64
    lookup, n_slabs, n_chunks = _make_lookup(chunk)
    idx = enc_tokens.astype(jnp.int32).reshape(B, n_slabs, n_chunks, chunk)
    return lookup(emb_table, idx)
